# baseline (speedup 1.0000x reference)
import functools

import jax
import jax.numpy as jnp
from jax import lax
from jax.experimental import pallas as pl
from jax.experimental.pallas import tpu as pltpu
from jax.experimental.pallas import tpu_sc as plsc

DROPOUT = 0.1
NC = 2
NS = 16
NW = NC * NS
LANES = 16

VOCAB = 1000000
D = 32
B = 16384
H = 50
BPW = B // NW
NTILE_E = D // 8
SLAB = NTILE_E * (B // 128) * 1024


def _sc_embed_dropout(weight, scale, idx_t):
    mesh = plsc.VectorSubcoreMesh(core_axis_name="c", subcore_axis_name="s")

    @functools.partial(
        pl.kernel,
        out_type=jax.ShapeDtypeStruct((H * SLAB,), jnp.float32),
        mesh=mesh,
        scratch_types=[
            pltpu.VMEM((H, BPW), jnp.int32),
            pltpu.VMEM((2, BPW), jnp.float32),
            pltpu.VMEM((2, BPW, D), jnp.float32),
            pltpu.VMEM((BPW * D,), jnp.float32),
            pltpu.SemaphoreType.DMA,
            pltpu.SemaphoreType.DMA,
            pltpu.SemaphoreType.DMA,
        ],
        compiler_params=pltpu.CompilerParams(
            use_tc_tiling_on_sc=False, needs_layout_passes=False),
    )
    def run(tbl, scale_hbm, idx_hbm, out_hbm, idx_all, sc_v, rows_v, obuf,
            sem_w, sem_s, sem_o):
        wid = lax.axis_index("s") * NC + lax.axis_index("c")
        b0 = wid * BPW
        cb0 = b0 // 128
        ncb = BPW // 128
        lanes = jax.lax.iota(jnp.int32, LANES)
        perms = [lanes ^ s for s in (1, 2, 4, 8)]
        masks = [(lanes & s) == 0 for s in (1, 2, 4, 8)]
        gdn = lax.GatherDimensionNumbers(
            offset_dims=(), collapsed_slice_dims=(0,), start_index_map=(0,))

        def perm(x, ix):
            return lax.gather(x, ix[:, None], gdn, slice_sizes=(1,),
                              mode=lax.GatherScatterMode.PROMISE_IN_BOUNDS)

        def out_copies(h):
            base_o = h * SLAB + cb0 * 1024
            return [
                pltpu.make_async_copy(
                    obuf.at[pl.ds(re * (ncb * 1024), ncb * 1024)],
                    out_hbm.at[pl.ds(base_o + re * (B // 128) * 1024,
                                     ncb * 1024)],
                    sem_o)
                for re in range(NTILE_E)]

        pltpu.sync_copy(idx_hbm.at[:, pl.ds(b0, BPW)], idx_all)
        pltpu.async_copy(tbl.at[idx_all.at[0]], rows_v.at[0], sem_w).start()
        pltpu.async_copy(scale_hbm.at[idx_all.at[0]], sc_v.at[0], sem_s).start()

        def h_body(h, carry):
            p = h % 2
            pltpu.make_async_copy(tbl.at[idx_all.at[h]], rows_v.at[p],
                                  sem_w).wait()
            pltpu.make_async_copy(scale_hbm.at[idx_all.at[h]], sc_v.at[p],
                                  sem_s).wait()

            @pl.when(h + 1 < H)
            def _():
                hn = jnp.minimum(h + 1, H - 1)
                pltpu.async_copy(tbl.at[idx_all.at[hn]], rows_v.at[1 - p],
                                 sem_w).start()
                pltpu.async_copy(scale_hbm.at[idx_all.at[hn]],
                                 sc_v.at[1 - p], sem_s).start()

            @pl.when(h > 0)
            def _():
                for cp in out_copies(h):
                    cp.wait()

            @plsc.parallel_loop(0, BPW // LANES, 1, unroll=1)
            def g_body(g):
                bl = g * LANES
                scvec = sc_v[p, pl.ds(bl, LANES)]
                base = (bl // 128) * 1024 + (bl % 128)
                for half in range(D // LANES):
                    v = [rows_v[p, bl + j, pl.ds(half * LANES, LANES)]
                         for j in range(LANES)]
                    for st, s in enumerate((1, 2, 4, 8)):
                        ix, m = perms[st], masks[st]
                        for i in range(LANES):
                            if i & s:
                                continue
                            jj = i | s
                            a, b = v[i], v[jj]
                            ax = perm(a, ix)
                            bx = perm(b, ix)
                            v[i] = jnp.where(m, a, bx)
                            v[jj] = jnp.where(m, ax, b)
                    for el in range(LANES):
                        e = half * LANES + el
                        dst = ((e // 8) * (ncb * 1024) + (e % 8) * 128 + base)
                        obuf[pl.ds(dst, LANES)] = v[el] * scvec

            for cp in out_copies(h):
                cp.start()
            return carry

        lax.fori_loop(0, H, h_body, 0)
        for cp in out_copies(H - 1):
            cp.wait()

    return run(weight, scale, idx_t)


def kernel(weight, words):
    mask_key = jax.random.fold_in(jax.random.key(0), 1)
    keep = jax.random.bernoulli(
        mask_key, 1.0 - DROPOUT, (VOCAB, 1)).astype(weight.dtype)
    scale = (keep / (1.0 - DROPOUT)).reshape(VOCAB)

    idx_t = words.T.astype(jnp.int32)
    out_flat = _sc_embed_dropout(weight, scale, idx_t)
    t = out_flat.reshape(H, NTILE_E, B // 128, 8, 128)
    out = t.transpose(2, 4, 0, 1, 3).reshape(B, H, D)
    return out


# speedup vs baseline: 11.0130x; 11.0130x over previous
import functools

import jax
import jax.numpy as jnp
from jax import lax
from jax.experimental import pallas as pl
from jax.experimental.pallas import tpu as pltpu
from jax.experimental.pallas import tpu_sc as plsc

DROPOUT = 0.1
NC = 2
NS = 16
NW = NC * NS
LANES = 16

VOCAB = 1000000
D = 32
B = 16384
H = 50
BPW = B // NW
NTILE_E = D // 8
SLAB = NTILE_E * (B // 128) * 1024


def _sc_embed_dropout(weight, scale, idx_t):
    mesh = plsc.VectorSubcoreMesh(core_axis_name="c", subcore_axis_name="s")

    @functools.partial(
        pl.kernel,
        out_type=jax.ShapeDtypeStruct((H * SLAB,), jnp.float32),
        mesh=mesh,
        scratch_types=[
            pltpu.VMEM((H, BPW), jnp.int32),
            pltpu.VMEM((BPW,), jnp.float32),
            pltpu.VMEM((BPW, D), jnp.float32),
            pltpu.VMEM((BPW * D,), jnp.float32),
            pltpu.SemaphoreType.DMA,
            pltpu.SemaphoreType.DMA,
            pltpu.SemaphoreType.DMA,
        ],
        compiler_params=pltpu.CompilerParams(
            use_tc_tiling_on_sc=False, needs_layout_passes=False),
    )
    def run(tbl, scale_hbm, idx_hbm, out_hbm, idx_all, sc_v, rows_v, obuf,
            sem_w, sem_s, sem_o):
        wid = lax.axis_index("s") * NC + lax.axis_index("c")
        b0 = wid * BPW
        cb0 = b0 // 128
        ncb = BPW // 128
        lanes = jax.lax.iota(jnp.int32, LANES)
        perms = [lanes ^ s for s in (1, 2, 4, 8)]
        masks = [(lanes & s) == 0 for s in (1, 2, 4, 8)]
        gdn = lax.GatherDimensionNumbers(
            offset_dims=(), collapsed_slice_dims=(0,), start_index_map=(0,))

        def perm(x, ix):
            return lax.gather(x, ix[:, None], gdn, slice_sizes=(1,),
                              mode=lax.GatherScatterMode.PROMISE_IN_BOUNDS)

        def out_copies(h):
            base_o = h * SLAB + cb0 * 1024
            return [
                pltpu.make_async_copy(
                    obuf.at[pl.ds(re * (ncb * 1024), ncb * 1024)],
                    out_hbm.at[pl.ds(base_o + re * (B // 128) * 1024,
                                     ncb * 1024)],
                    sem_o)
                for re in range(NTILE_E)]

        pltpu.sync_copy(idx_hbm.at[:, pl.ds(b0, BPW)], idx_all)

        def h_body(h, carry):
            idx_v = idx_all.at[h]
            cp_w = pltpu.async_copy(tbl.at[idx_v], rows_v, sem_w)
            cp_s = pltpu.async_copy(scale_hbm.at[idx_v], sc_v, sem_s)
            cp_w.wait()
            cp_s.wait()

            @pl.when(h > 0)
            def _():
                for cp in out_copies(h):
                    cp.wait()

            @plsc.parallel_loop(0, BPW // LANES, 1, unroll=1)
            def g_body(g):
                bl = g * LANES
                scvec = sc_v[pl.ds(bl, LANES)]
                base = (bl // 128) * 1024 + (bl % 128)
                for half in range(D // LANES):
                    v = [rows_v[bl + j, pl.ds(half * LANES, LANES)]
                         for j in range(LANES)]
                    for st, s in enumerate((1, 2, 4, 8)):
                        ix, m = perms[st], masks[st]
                        for i in range(LANES):
                            if i & s:
                                continue
                            jj = i | s
                            a, b = v[i], v[jj]
                            ax = perm(a, ix)
                            bx = perm(b, ix)
                            v[i] = jnp.where(m, a, bx)
                            v[jj] = jnp.where(m, ax, b)
                    for el in range(LANES):
                        e = half * LANES + el
                        dst = ((e // 8) * (ncb * 1024) + (e % 8) * 128 + base)
                        obuf[pl.ds(dst, LANES)] = v[el] * scvec

            for cp in out_copies(h):
                cp.start()
            return carry

        lax.fori_loop(0, H, h_body, 0)
        for cp in out_copies(H - 1):
            cp.wait()

    return run(weight, scale, idx_t)


def kernel(weight, words):
    mask_key = jax.random.fold_in(jax.random.key(0), 1)
    keep = jax.random.bernoulli(
        mask_key, 1.0 - DROPOUT, (VOCAB, 1)).astype(weight.dtype)
    scale = (keep / (1.0 - DROPOUT)).reshape(VOCAB)

    idx_t = words.T.astype(jnp.int32)
    out_flat = _sc_embed_dropout(weight, scale, idx_t)
    t = out_flat.reshape(H, NTILE_E, B // 128, 8, 128)
    out = t.transpose(2, 4, 0, 1, 3).reshape(B, H, D)
    return out
